# initial kernel scaffold (unmeasured)
import jax
import jax.numpy as jnp
from jax import lax
from jax.experimental import pallas as pl
from jax.experimental.pallas import tpu as pltpu

N_DEV = 4
B, SQ, D = 4, 256, 1024
HQ_LOCAL = 8
DH = 128
SKV = 1024
SCALE = 0.08838834764831843


def kernel(x, Wq, Wo, K_ext, V_ext):
    i = lax.axis_index("i")
    K_loc = lax.dynamic_slice_in_dim(K_ext, 2 * i, 2, axis=2)
    V_loc = lax.dynamic_slice_in_dim(V_ext, 2 * i, 2, axis=2)
    K_loc = jnp.transpose(K_loc, (0, 2, 1, 3))
    V_loc = jnp.transpose(V_loc, (0, 2, 1, 3))
    x2 = x.reshape(B * SQ, D)

    def body(x_ref, wq_ref, wo_ref, k_ref, v_ref, out_ref,
             q_ref, attn_ref, comm_ref, send_sems, recv_sems):
        my = lax.axis_index("i")
        left = lax.rem(my + N_DEV - 1, N_DEV)
        right = lax.rem(my + 1, N_DEV)

        barrier = pltpu.get_barrier_semaphore()
        for nbr in (left, right):
            pl.semaphore_signal(barrier, inc=1, device_id=(nbr,),
                                device_id_type=pl.DeviceIdType.MESH)
        pl.semaphore_wait(barrier, 2)

        q_ref[...] = jnp.dot(x_ref[...], wq_ref[...],
                             preferred_element_type=jnp.float32)

        for b in range(B):
            for h in range(HQ_LOCAL):
                kv = h // 4
                qblk = q_ref[b * SQ:(b + 1) * SQ, h * DH:(h + 1) * DH]
                kblk = k_ref[b, kv]
                vblk = v_ref[b, kv]
                s = lax.dot_general(
                    qblk, kblk, (((1,), (1,)), ((), ())),
                    preferred_element_type=jnp.float32) * SCALE
                m = jnp.max(s, axis=1, keepdims=True)
                p = jnp.exp(s - m)
                l = jnp.sum(p, axis=1, keepdims=True)
                o = jnp.dot(p, vblk, preferred_element_type=jnp.float32) / l
                attn_ref[b * SQ:(b + 1) * SQ, h * DH:(h + 1) * DH] = o

        comm_ref[0, :, :] = jnp.dot(attn_ref[...], wo_ref[...],
                                    preferred_element_type=jnp.float32)
        out_ref[...] = comm_ref[0, :, :]

        for hop in range(N_DEV - 1):
            rdma = pltpu.make_async_remote_copy(
                src_ref=comm_ref.at[hop],
                dst_ref=comm_ref.at[hop + 1],
                send_sem=send_sems.at[hop],
                recv_sem=recv_sems.at[hop],
                device_id=(right,),
                device_id_type=pl.DeviceIdType.MESH,
            )
            rdma.start()
            rdma.wait()
            out_ref[...] += comm_ref[hop + 1, :, :]

    out = pl.pallas_call(
        body,
        out_shape=jax.ShapeDtypeStruct((B * SQ, D), jnp.float32),
        in_specs=[pl.BlockSpec(memory_space=pltpu.VMEM)] * 5,
        out_specs=pl.BlockSpec(memory_space=pltpu.VMEM),
        scratch_shapes=[
            pltpu.VMEM((B * SQ, D), jnp.float32),
            pltpu.VMEM((B * SQ, D), jnp.float32),
            pltpu.VMEM((N_DEV, B * SQ, D), jnp.float32),
            pltpu.SemaphoreType.DMA((N_DEV - 1,)),
            pltpu.SemaphoreType.DMA((N_DEV - 1,)),
        ],
        compiler_params=pltpu.CompilerParams(collective_id=0),
    )(x2, Wq, Wo, K_loc, V_loc)
    return out.reshape(B, SQ, D)


# baseline (device time: 208753 ns/iter reference)
import jax
import jax.numpy as jnp
from jax import lax
from jax.experimental import pallas as pl
from jax.experimental.pallas import tpu as pltpu

N_DEV = 4
B, SQ, D = 4, 256, 1024
HQ_LOCAL = 8
DH = 128
SKV = 1024
SCALE = 0.08838834764831843

Q_SLOT = 2
ATTN_SLOT = 3


def kernel(x, Wq, Wo, K_ext, V_ext):
    i = lax.axis_index("i")
    K_loc = lax.dynamic_slice_in_dim(K_ext, 2 * i, 2, axis=2)
    V_loc = lax.dynamic_slice_in_dim(V_ext, 2 * i, 2, axis=2)
    K_loc = jnp.transpose(K_loc, (0, 2, 1, 3)).reshape(B * 2, SKV, DH)
    V_loc = jnp.transpose(V_loc, (0, 2, 1, 3)).reshape(B * 2, SKV, DH)
    x2 = x.reshape(B * SQ, D)

    def body(x_ref, wq_ref, wo_ref, k_ref, v_ref, out_ref,
             comm_ref, send_sems, recv_sems):
        my = lax.axis_index("i")
        left = lax.rem(my + N_DEV - 1, N_DEV)
        right = lax.rem(my + 1, N_DEV)

        barrier = pltpu.get_barrier_semaphore()
        for nbr in (left, right):
            pl.semaphore_signal(barrier, inc=1, device_id=(nbr,),
                                device_id_type=pl.DeviceIdType.MESH)
        pl.semaphore_wait(barrier, 2)

        comm_ref[Q_SLOT, :, :] = jnp.dot(x_ref[...], wq_ref[...],
                                         preferred_element_type=jnp.float32)

        def attn_step(t, carry):
            b = t // HQ_LOCAL
            h = t % HQ_LOCAL
            kvi = b * 2 + h // 4
            qblk = comm_ref[Q_SLOT, pl.ds(b * SQ, SQ), pl.ds(h * DH, DH)]
            kblk = k_ref[kvi]
            vblk = v_ref[kvi]
            s = lax.dot_general(
                qblk, kblk, (((1,), (1,)), ((), ())),
                preferred_element_type=jnp.float32) * SCALE
            m = jnp.max(s, axis=1, keepdims=True)
            p = jnp.exp(s - m)
            l = jnp.sum(p, axis=1, keepdims=True)
            o = jnp.dot(p, vblk, preferred_element_type=jnp.float32) / l
            comm_ref[ATTN_SLOT, pl.ds(b * SQ, SQ), pl.ds(h * DH, DH)] = o
            return carry

        lax.fori_loop(0, B * HQ_LOCAL, attn_step, 0)

        comm_ref[0, :, :] = jnp.dot(comm_ref[ATTN_SLOT, :, :], wo_ref[...],
                                    preferred_element_type=jnp.float32)
        out_ref[...] = comm_ref[0, :, :]

        for hop in range(N_DEV - 1):
            rdma = pltpu.make_async_remote_copy(
                src_ref=comm_ref.at[hop],
                dst_ref=comm_ref.at[hop + 1],
                send_sem=send_sems.at[hop],
                recv_sem=recv_sems.at[hop],
                device_id=(right,),
                device_id_type=pl.DeviceIdType.MESH,
            )
            rdma.start()
            rdma.wait()
            out_ref[...] += comm_ref[hop + 1, :, :]

    out = pl.pallas_call(
        body,
        out_shape=jax.ShapeDtypeStruct((B * SQ, D), jnp.float32),
        in_specs=[pl.BlockSpec(memory_space=pltpu.VMEM)] * 5,
        out_specs=pl.BlockSpec(memory_space=pltpu.VMEM),
        scratch_shapes=[
            pltpu.VMEM((N_DEV, B * SQ, D), jnp.float32),
            pltpu.SemaphoreType.DMA((N_DEV - 1,)),
            pltpu.SemaphoreType.DMA((N_DEV - 1,)),
        ],
        compiler_params=pltpu.CompilerParams(
            collective_id=0, vmem_limit_bytes=100 * 1024 * 1024),
    )(x2, Wq, Wo, K_loc, V_loc)
    return out.reshape(B, SQ, D)


# device time: 145164 ns/iter; 1.4380x vs baseline; 1.4380x over previous
import jax
import jax.numpy as jnp
from jax import lax
from jax.experimental import pallas as pl
from jax.experimental.pallas import tpu as pltpu

N_DEV = 4
B, SQ, D = 4, 256, 1024
HQ_LOCAL = 8
DH = 128
SKV = 1024
SCALE = 0.08838834764831843
CHUNK = (B * SQ) // N_DEV


def kernel(x, Wq, Wo, K_ext, V_ext):
    i = lax.axis_index("i")
    K_loc = lax.dynamic_slice_in_dim(K_ext, 2 * i, 2, axis=2)
    V_loc = lax.dynamic_slice_in_dim(V_ext, 2 * i, 2, axis=2)
    K_loc = jnp.transpose(K_loc, (0, 2, 1, 3)).reshape(B * 2, SKV, DH)
    V_loc = jnp.transpose(V_loc, (0, 2, 1, 3)).reshape(B * 2, SKV, DH)
    x2 = x.reshape(B * SQ, D)

    def body(x_ref, wq_ref, wo_ref, k_ref, v_ref, out_ref,
             q_ref, attn_ref, rs_buf,
             rs_send, rs_recv, ag_send, ag_recv):
        my = lax.axis_index("i")
        left = lax.rem(my + N_DEV - 1, N_DEV)
        right = lax.rem(my + 1, N_DEV)

        barrier = pltpu.get_barrier_semaphore()
        for nbr in (left, right):
            pl.semaphore_signal(barrier, inc=1, device_id=(nbr,),
                                device_id_type=pl.DeviceIdType.MESH)
        pl.semaphore_wait(barrier, 2)

        q_ref[...] = jnp.dot(x_ref[...], wq_ref[...],
                             preferred_element_type=jnp.float32)

        def attn_step(t, carry):
            b = t // HQ_LOCAL
            h = t % HQ_LOCAL
            kvi = b * 2 + h // 4
            qblk = q_ref[pl.ds(b * SQ, SQ), pl.ds(h * DH, DH)]
            kblk = k_ref[kvi]
            vblk = v_ref[kvi]
            s = lax.dot_general(
                qblk, kblk, (((1,), (1,)), ((), ())),
                preferred_element_type=jnp.float32) * SCALE
            m = jnp.max(s, axis=1, keepdims=True)
            p = jnp.exp(s - m)
            l = jnp.sum(p, axis=1, keepdims=True)
            o = jnp.dot(p, vblk, preferred_element_type=jnp.float32) / l
            attn_ref[pl.ds(b * SQ, SQ), pl.ds(h * DH, DH)] = o
            return carry

        lax.fori_loop(0, B * HQ_LOCAL, attn_step, 0)

        out_ref[...] = jnp.dot(attn_ref[...], wo_ref[...],
                               preferred_element_type=jnp.float32)

        for s in range(N_DEV - 1):
            c_send = lax.rem(my - s + N_DEV, N_DEV)
            rdma = pltpu.make_async_remote_copy(
                src_ref=out_ref.at[pl.ds(c_send * CHUNK, CHUNK)],
                dst_ref=rs_buf.at[s],
                send_sem=rs_send.at[s],
                recv_sem=rs_recv.at[s],
                device_id=(right,),
                device_id_type=pl.DeviceIdType.MESH,
            )
            rdma.start()
            rdma.wait()
            c_recv = lax.rem(my - 1 - s + N_DEV, N_DEV)
            rows = pl.ds(c_recv * CHUNK, CHUNK)
            out_ref[rows, :] = out_ref[rows, :] + rs_buf[s]

        for s in range(N_DEV - 1):
            c = lax.rem(my + 1 - s + N_DEV, N_DEV)
            rows = pl.ds(c * CHUNK, CHUNK)
            rdma = pltpu.make_async_remote_copy(
                src_ref=out_ref.at[rows],
                dst_ref=out_ref.at[rows],
                send_sem=ag_send.at[s],
                recv_sem=ag_recv.at[s],
                device_id=(right,),
                device_id_type=pl.DeviceIdType.MESH,
            )
            rdma.start()
            rdma.wait()

    out = pl.pallas_call(
        body,
        out_shape=jax.ShapeDtypeStruct((B * SQ, D), jnp.float32),
        in_specs=[pl.BlockSpec(memory_space=pltpu.VMEM)] * 5,
        out_specs=pl.BlockSpec(memory_space=pltpu.VMEM),
        scratch_shapes=[
            pltpu.VMEM((B * SQ, D), jnp.float32),
            pltpu.VMEM((B * SQ, D), jnp.float32),
            pltpu.VMEM((N_DEV - 1, CHUNK, D), jnp.float32),
            pltpu.SemaphoreType.DMA((N_DEV - 1,)),
            pltpu.SemaphoreType.DMA((N_DEV - 1,)),
            pltpu.SemaphoreType.DMA((N_DEV - 1,)),
            pltpu.SemaphoreType.DMA((N_DEV - 1,)),
        ],
        compiler_params=pltpu.CompilerParams(
            collective_id=0, vmem_limit_bytes=100 * 1024 * 1024),
    )(x2, Wq, Wo, K_loc, V_loc)
    return out.reshape(B, SQ, D)


# device time: 106703 ns/iter; 1.9564x vs baseline; 1.3604x over previous
import jax
import jax.numpy as jnp
from jax import lax
from jax.experimental import pallas as pl
from jax.experimental.pallas import tpu as pltpu

N_DEV = 4
B, SQ, D = 4, 256, 1024
HQ_LOCAL = 8
DH = 128
SKV = 1024
SCALE = 0.08838834764831843


def kernel(x, Wq, Wo, K_ext, V_ext):
    i = lax.axis_index("i")
    K_loc = lax.dynamic_slice_in_dim(K_ext, 2 * i, 2, axis=2)
    V_loc = lax.dynamic_slice_in_dim(V_ext, 2 * i, 2, axis=2)
    K_loc = jnp.transpose(K_loc, (0, 2, 1, 3)).reshape(B * 2, SKV, DH)
    V_loc = jnp.transpose(V_loc, (0, 2, 1, 3)).reshape(B * 2, SKV, DH)
    x2 = x.reshape(B * SQ, D)

    def body(x_ref, wq_ref, wo_ref, k_ref, v_ref, out_ref,
             q_chunk, attn_chunk, rs_buf, rs_send, rs_recv, ag_sems):
        my = lax.axis_index("i")
        left = lax.rem(my + N_DEV - 1, N_DEV)
        right = lax.rem(my + 1, N_DEV)

        barrier = pltpu.get_barrier_semaphore()
        for nbr in (left, right):
            pl.semaphore_signal(barrier, inc=1, device_id=(nbr,),
                                device_id_type=pl.DeviceIdType.MESH)
        pl.semaphore_wait(barrier, 2)

        def compute_chunk(b):
            rows = pl.ds(b * SQ, SQ)
            q_chunk[...] = jnp.dot(x_ref[rows, :], wq_ref[...],
                                   preferred_element_type=jnp.float32)

            def attn_step(h, carry):
                kvi = b * 2 + h // 4
                qblk = q_chunk[:, pl.ds(h * DH, DH)]
                kblk = k_ref[kvi]
                vblk = v_ref[kvi]
                s = lax.dot_general(
                    qblk, kblk, (((1,), (1,)), ((), ())),
                    preferred_element_type=jnp.float32) * SCALE
                m = jnp.max(s, axis=1, keepdims=True)
                p = jnp.exp(s - m)
                l = jnp.sum(p, axis=1, keepdims=True)
                o = jnp.dot(p, vblk, preferred_element_type=jnp.float32) / l
                attn_chunk[:, pl.ds(h * DH, DH)] = o
                return carry

            lax.fori_loop(0, HQ_LOCAL, attn_step, 0)
            out_ref[rows, :] = jnp.dot(attn_chunk[...], wo_ref[...],
                                       preferred_element_type=jnp.float32)

        compute_chunk(my)
        prev = pltpu.make_async_remote_copy(
            src_ref=out_ref.at[pl.ds(my * SQ, SQ)],
            dst_ref=rs_buf.at[0],
            send_sem=rs_send.at[0], recv_sem=rs_recv.at[0],
            device_id=(right,), device_id_type=pl.DeviceIdType.MESH,
        )
        prev.start()
        for idx in range(1, N_DEV):
            b = lax.rem(my - idx + N_DEV, N_DEV)
            compute_chunk(b)
            prev.wait()
            rows = pl.ds(b * SQ, SQ)
            out_ref[rows, :] = out_ref[rows, :] + rs_buf[idx - 1]
            if idx < N_DEV - 1:
                prev = pltpu.make_async_remote_copy(
                    src_ref=out_ref.at[rows],
                    dst_ref=rs_buf.at[idx],
                    send_sem=rs_send.at[idx], recv_sem=rs_recv.at[idx],
                    device_id=(right,), device_id_type=pl.DeviceIdType.MESH,
                )
                prev.start()

        c_own = lax.rem(my + 1, N_DEV)
        rows_own = pl.ds(c_own * SQ, SQ)
        a_r = pltpu.make_async_remote_copy(
            src_ref=out_ref.at[rows_own], dst_ref=out_ref.at[rows_own],
            send_sem=ag_sems.at[0], recv_sem=ag_sems.at[1],
            device_id=(right,), device_id_type=pl.DeviceIdType.MESH,
        )
        a_l = pltpu.make_async_remote_copy(
            src_ref=out_ref.at[rows_own], dst_ref=out_ref.at[rows_own],
            send_sem=ag_sems.at[2], recv_sem=ag_sems.at[3],
            device_id=(left,), device_id_type=pl.DeviceIdType.MESH,
        )
        a_r.start()
        a_l.start()
        a_r.wait()
        rows_my = pl.ds(my * SQ, SQ)
        fwd = pltpu.make_async_remote_copy(
            src_ref=out_ref.at[rows_my], dst_ref=out_ref.at[rows_my],
            send_sem=ag_sems.at[4], recv_sem=ag_sems.at[5],
            device_id=(right,), device_id_type=pl.DeviceIdType.MESH,
        )
        fwd.start()
        a_l.wait()
        fwd.wait()

    out = pl.pallas_call(
        body,
        out_shape=jax.ShapeDtypeStruct((B * SQ, D), jnp.float32),
        in_specs=[pl.BlockSpec(memory_space=pltpu.VMEM)] * 5,
        out_specs=pl.BlockSpec(memory_space=pltpu.VMEM),
        scratch_shapes=[
            pltpu.VMEM((SQ, D), jnp.float32),
            pltpu.VMEM((SQ, D), jnp.float32),
            pltpu.VMEM((N_DEV - 1, SQ, D), jnp.float32),
            pltpu.SemaphoreType.DMA((N_DEV - 1,)),
            pltpu.SemaphoreType.DMA((N_DEV - 1,)),
            pltpu.SemaphoreType.DMA((6,)),
        ],
        compiler_params=pltpu.CompilerParams(
            collective_id=0, vmem_limit_bytes=100 * 1024 * 1024),
    )(x2, Wq, Wo, K_loc, V_loc)
    return out.reshape(B, SQ, D)


# device time: 90846 ns/iter; 2.2979x vs baseline; 1.1745x over previous
import jax
import jax.numpy as jnp
from jax import lax
from jax.experimental import pallas as pl
from jax.experimental.pallas import tpu as pltpu

N_DEV = 4
B, SQ, D = 4, 256, 1024
HQ_LOCAL = 8
DH = 128
SKV = 1024
SCALE = 0.08838834764831843
BF = jnp.bfloat16


def kernel(x, Wq, Wo, K_ext, V_ext):
    x2 = x.reshape(B * SQ, D)

    def body(x_ref, wq_ref, wo_ref, k_any, v_any, out_ref,
             x_bf, wq_bf, wo_bf, k_heads, v_heads, k_bf, v_bf,
             q_bf, attn_bf, rs_buf,
             kv_sems, rs_send, rs_recv, ag_sems):
        my = lax.axis_index("i")
        left = lax.rem(my + N_DEV - 1, N_DEV)
        right = lax.rem(my + 1, N_DEV)

        barrier = pltpu.get_barrier_semaphore()
        for nbr in (left, right):
            pl.semaphore_signal(barrier, inc=1, device_id=(nbr,),
                                device_id_type=pl.DeviceIdType.MESH)
        pl.semaphore_wait(barrier, 2)

        for b in range(B):
            for kvl in range(2):
                g = 2 * my + kvl
                pltpu.make_async_copy(
                    k_any.at[b, :, g, :], k_heads.at[b * 2 + kvl],
                    kv_sems.at[b, kvl]).start()
                pltpu.make_async_copy(
                    v_any.at[b, :, g, :], v_heads.at[b * 2 + kvl],
                    kv_sems.at[b, 2 + kvl]).start()

        x_bf[...] = x_ref[...].astype(BF)
        wq_bf[...] = wq_ref[...].astype(BF)
        wo_bf[...] = wo_ref[...].astype(BF)

        def compute_chunk(b):
            rows = pl.ds(b * SQ, SQ)
            q_bf[...] = jnp.dot(x_bf[rows, :], wq_bf[...],
                                preferred_element_type=jnp.float32).astype(BF)

            for kvl in range(2):
                g = 2 * my + kvl
                pltpu.make_async_copy(
                    k_any.at[b, :, g, :], k_heads.at[b * 2 + kvl],
                    kv_sems.at[b, kvl]).wait()
                pltpu.make_async_copy(
                    v_any.at[b, :, g, :], v_heads.at[b * 2 + kvl],
                    kv_sems.at[b, 2 + kvl]).wait()
                k_bf[kvl] = k_heads[b * 2 + kvl].astype(BF)
                v_bf[kvl] = v_heads[b * 2 + kvl].astype(BF)

            def attn_step(h, carry):
                kvl = h // 4
                qblk = q_bf[:, pl.ds(h * DH, DH)]
                s = lax.dot_general(
                    qblk, k_bf[kvl], (((1,), (1,)), ((), ())),
                    preferred_element_type=jnp.float32) * SCALE
                m = jnp.max(s, axis=1, keepdims=True)
                p = jnp.exp(s - m)
                l = jnp.sum(p, axis=1, keepdims=True)
                o = jnp.dot(p.astype(BF), v_bf[kvl],
                            preferred_element_type=jnp.float32) / l
                attn_bf[:, pl.ds(h * DH, DH)] = o.astype(BF)
                return carry

            lax.fori_loop(0, HQ_LOCAL, attn_step, 0)
            out_ref[rows, :] = jnp.dot(attn_bf[...], wo_bf[...],
                                       preferred_element_type=jnp.float32)

        compute_chunk(my)
        prev = pltpu.make_async_remote_copy(
            src_ref=out_ref.at[pl.ds(my * SQ, SQ)],
            dst_ref=rs_buf.at[0],
            send_sem=rs_send.at[0], recv_sem=rs_recv.at[0],
            device_id=(right,), device_id_type=pl.DeviceIdType.MESH,
        )
        prev.start()
        for idx in range(1, N_DEV):
            b = lax.rem(my - idx + N_DEV, N_DEV)
            compute_chunk(b)
            prev.wait()
            rows = pl.ds(b * SQ, SQ)
            out_ref[rows, :] = out_ref[rows, :] + rs_buf[idx - 1]
            if idx < N_DEV - 1:
                prev = pltpu.make_async_remote_copy(
                    src_ref=out_ref.at[rows],
                    dst_ref=rs_buf.at[idx],
                    send_sem=rs_send.at[idx], recv_sem=rs_recv.at[idx],
                    device_id=(right,), device_id_type=pl.DeviceIdType.MESH,
                )
                prev.start()

        c_own = lax.rem(my + 1, N_DEV)
        rows_own = pl.ds(c_own * SQ, SQ)
        a_r = pltpu.make_async_remote_copy(
            src_ref=out_ref.at[rows_own], dst_ref=out_ref.at[rows_own],
            send_sem=ag_sems.at[0], recv_sem=ag_sems.at[1],
            device_id=(right,), device_id_type=pl.DeviceIdType.MESH,
        )
        a_l = pltpu.make_async_remote_copy(
            src_ref=out_ref.at[rows_own], dst_ref=out_ref.at[rows_own],
            send_sem=ag_sems.at[2], recv_sem=ag_sems.at[3],
            device_id=(left,), device_id_type=pl.DeviceIdType.MESH,
        )
        a_r.start()
        a_l.start()
        a_r.wait()
        rows_my = pl.ds(my * SQ, SQ)
        fwd = pltpu.make_async_remote_copy(
            src_ref=out_ref.at[rows_my], dst_ref=out_ref.at[rows_my],
            send_sem=ag_sems.at[4], recv_sem=ag_sems.at[5],
            device_id=(right,), device_id_type=pl.DeviceIdType.MESH,
        )
        fwd.start()
        a_l.wait()
        fwd.wait()

    out = pl.pallas_call(
        body,
        out_shape=jax.ShapeDtypeStruct((B * SQ, D), jnp.float32),
        in_specs=[
            pl.BlockSpec(memory_space=pltpu.VMEM),
            pl.BlockSpec(memory_space=pltpu.VMEM),
            pl.BlockSpec(memory_space=pltpu.VMEM),
            pl.BlockSpec(memory_space=pl.ANY),
            pl.BlockSpec(memory_space=pl.ANY),
        ],
        out_specs=pl.BlockSpec(memory_space=pltpu.VMEM),
        scratch_shapes=[
            pltpu.VMEM((B * SQ, D), BF),
            pltpu.VMEM((D, D), BF),
            pltpu.VMEM((D, D), BF),
            pltpu.VMEM((B * 2, SKV, DH), jnp.float32),
            pltpu.VMEM((B * 2, SKV, DH), jnp.float32),
            pltpu.VMEM((2, SKV, DH), BF),
            pltpu.VMEM((2, SKV, DH), BF),
            pltpu.VMEM((SQ, D), BF),
            pltpu.VMEM((SQ, D), BF),
            pltpu.VMEM((N_DEV - 1, SQ, D), jnp.float32),
            pltpu.SemaphoreType.DMA((B, 4)),
            pltpu.SemaphoreType.DMA((N_DEV - 1,)),
            pltpu.SemaphoreType.DMA((N_DEV - 1,)),
            pltpu.SemaphoreType.DMA((6,)),
        ],
        compiler_params=pltpu.CompilerParams(
            collective_id=0, vmem_limit_bytes=100 * 1024 * 1024),
    )(x2, Wq, Wo, K_ext, V_ext)
    return out.reshape(B, SQ, D)


# device time: 78333 ns/iter; 2.6649x vs baseline; 1.1597x over previous
import jax
import jax.numpy as jnp
from jax import lax
from jax.experimental import pallas as pl
from jax.experimental.pallas import tpu as pltpu

N_DEV = 4
B, SQ, D = 4, 256, 1024
HQ_LOCAL = 8
DH = 128
SKV = 1024
SCALE = 0.08838834764831843
BF = jnp.bfloat16


def kernel(x, Wq, Wo, K_ext, V_ext):
    x2 = x.reshape(B * SQ, D)

    def body(x_ref, wq_ref, wo_ref, k_any, v_any, out_ref,
             x_bf, wq_bf, wo_bf, k_heads, v_heads, k_bf, v_bf,
             q_bf, attn_bf, rs_buf, ag_bf,
             kv_sems, rs_send, rs_recv, ag_sems):
        my = lax.axis_index("i")
        left = lax.rem(my + N_DEV - 1, N_DEV)
        right = lax.rem(my + 1, N_DEV)

        barrier = pltpu.get_barrier_semaphore()
        for nbr in (left, right):
            pl.semaphore_signal(barrier, inc=1, device_id=(nbr,),
                                device_id_type=pl.DeviceIdType.MESH)
        pl.semaphore_wait(barrier, 2)

        for b in range(B):
            for kvl in range(2):
                g = 2 * my + kvl
                pltpu.make_async_copy(
                    k_any.at[b, :, g, :], k_heads.at[b * 2 + kvl],
                    kv_sems.at[b, kvl]).start()
                pltpu.make_async_copy(
                    v_any.at[b, :, g, :], v_heads.at[b * 2 + kvl],
                    kv_sems.at[b, 2 + kvl]).start()

        x_bf[...] = x_ref[...].astype(BF)
        wq_bf[...] = wq_ref[...].astype(BF)
        wo_bf[...] = wo_ref[...].astype(BF)

        def compute_chunk(b):
            rows = pl.ds(b * SQ, SQ)
            q_bf[...] = (jnp.dot(x_bf[rows, :], wq_bf[...],
                                 preferred_element_type=jnp.float32)
                         * SCALE).astype(BF)

            for kvl in range(2):
                g = 2 * my + kvl
                pltpu.make_async_copy(
                    k_any.at[b, :, g, :], k_heads.at[b * 2 + kvl],
                    kv_sems.at[b, kvl]).wait()
                pltpu.make_async_copy(
                    v_any.at[b, :, g, :], v_heads.at[b * 2 + kvl],
                    kv_sems.at[b, 2 + kvl]).wait()
                k_bf[kvl] = k_heads[b * 2 + kvl].astype(BF)
                v_bf[kvl] = v_heads[b * 2 + kvl].astype(BF)

            def attn_step(h, carry):
                kvl = h // 4
                qblk = q_bf[:, pl.ds(h * DH, DH)]
                s = lax.dot_general(
                    qblk, k_bf[kvl], (((1,), (1,)), ((), ())),
                    preferred_element_type=jnp.float32)
                p = jnp.exp(s).astype(BF)
                l = jnp.sum(p.astype(jnp.float32), axis=1, keepdims=True)
                o = jnp.dot(p, v_bf[kvl],
                            preferred_element_type=jnp.float32) / l
                attn_bf[:, pl.ds(h * DH, DH)] = o.astype(BF)
                return carry

            lax.fori_loop(0, HQ_LOCAL, attn_step, 0)
            out_ref[rows, :] = jnp.dot(attn_bf[...], wo_bf[...],
                                       preferred_element_type=jnp.float32)

        compute_chunk(my)
        prev = pltpu.make_async_remote_copy(
            src_ref=out_ref.at[pl.ds(my * SQ, SQ)],
            dst_ref=rs_buf.at[0],
            send_sem=rs_send.at[0], recv_sem=rs_recv.at[0],
            device_id=(right,), device_id_type=pl.DeviceIdType.MESH,
        )
        prev.start()
        for idx in range(1, N_DEV):
            b = lax.rem(my - idx + N_DEV, N_DEV)
            compute_chunk(b)
            prev.wait()
            rows = pl.ds(b * SQ, SQ)
            out_ref[rows, :] = out_ref[rows, :] + rs_buf[idx - 1]
            if idx < N_DEV - 1:
                prev = pltpu.make_async_remote_copy(
                    src_ref=out_ref.at[rows],
                    dst_ref=rs_buf.at[idx],
                    send_sem=rs_send.at[idx], recv_sem=rs_recv.at[idx],
                    device_id=(right,), device_id_type=pl.DeviceIdType.MESH,
                )
                prev.start()

        c_own = lax.rem(my + 1, N_DEV)
        ag_bf[0] = out_ref[pl.ds(c_own * SQ, SQ), :].astype(BF)
        a_r = pltpu.make_async_remote_copy(
            src_ref=ag_bf.at[0], dst_ref=ag_bf.at[1],
            send_sem=ag_sems.at[0], recv_sem=ag_sems.at[1],
            device_id=(right,), device_id_type=pl.DeviceIdType.MESH,
        )
        a_l = pltpu.make_async_remote_copy(
            src_ref=ag_bf.at[0], dst_ref=ag_bf.at[2],
            send_sem=ag_sems.at[2], recv_sem=ag_sems.at[3],
            device_id=(left,), device_id_type=pl.DeviceIdType.MESH,
        )
        a_r.start()
        a_l.start()
        a_r.wait()
        fwd = pltpu.make_async_remote_copy(
            src_ref=ag_bf.at[1], dst_ref=ag_bf.at[3],
            send_sem=ag_sems.at[4], recv_sem=ag_sems.at[5],
            device_id=(right,), device_id_type=pl.DeviceIdType.MESH,
        )
        fwd.start()
        out_ref[pl.ds(my * SQ, SQ), :] = ag_bf[1].astype(jnp.float32)
        a_l.wait()
        c2 = lax.rem(my + 2, N_DEV)
        out_ref[pl.ds(c2 * SQ, SQ), :] = ag_bf[2].astype(jnp.float32)
        fwd.wait()
        c3 = lax.rem(my + 3, N_DEV)
        out_ref[pl.ds(c3 * SQ, SQ), :] = ag_bf[3].astype(jnp.float32)

    out = pl.pallas_call(
        body,
        out_shape=jax.ShapeDtypeStruct((B * SQ, D), jnp.float32),
        in_specs=[
            pl.BlockSpec(memory_space=pltpu.VMEM),
            pl.BlockSpec(memory_space=pltpu.VMEM),
            pl.BlockSpec(memory_space=pltpu.VMEM),
            pl.BlockSpec(memory_space=pl.ANY),
            pl.BlockSpec(memory_space=pl.ANY),
        ],
        out_specs=pl.BlockSpec(memory_space=pltpu.VMEM),
        scratch_shapes=[
            pltpu.VMEM((B * SQ, D), BF),
            pltpu.VMEM((D, D), BF),
            pltpu.VMEM((D, D), BF),
            pltpu.VMEM((B * 2, SKV, DH), jnp.float32),
            pltpu.VMEM((B * 2, SKV, DH), jnp.float32),
            pltpu.VMEM((2, SKV, DH), BF),
            pltpu.VMEM((2, SKV, DH), BF),
            pltpu.VMEM((SQ, D), BF),
            pltpu.VMEM((SQ, D), BF),
            pltpu.VMEM((N_DEV - 1, SQ, D), jnp.float32),
            pltpu.VMEM((N_DEV, SQ, D), BF),
            pltpu.SemaphoreType.DMA((B, 4)),
            pltpu.SemaphoreType.DMA((N_DEV - 1,)),
            pltpu.SemaphoreType.DMA((N_DEV - 1,)),
            pltpu.SemaphoreType.DMA((6,)),
        ],
        compiler_params=pltpu.CompilerParams(
            collective_id=0, vmem_limit_bytes=100 * 1024 * 1024),
    )(x2, Wq, Wo, K_ext, V_ext)
    return out.reshape(B, SQ, D)


# device time: 77609 ns/iter; 2.6898x vs baseline; 1.0093x over previous
import jax
import jax.numpy as jnp
from jax import lax
from jax.experimental import pallas as pl
from jax.experimental.pallas import tpu as pltpu

N_DEV = 4
B, SQ, D = 4, 256, 1024
HQ_LOCAL = 8
DH = 128
SKV = 1024
SCALE = 0.08838834764831843
BF = jnp.bfloat16


def kernel(x, Wq, Wo, K_ext, V_ext):
    x2 = x.reshape(B * SQ, D)

    def body(x_ref, wq_ref, wo_ref, k_any, v_any, out_ref,
             x_bf, wq_bf, wo_bf, k_heads, v_heads, k_bf, v_aug,
             q_bf, attn_bf, rs_buf, ag_bf,
             kv_sems, rs_send, rs_recv, ag_sems):
        my = lax.axis_index("i")
        left = lax.rem(my + N_DEV - 1, N_DEV)
        right = lax.rem(my + 1, N_DEV)

        barrier = pltpu.get_barrier_semaphore()
        for nbr in (left, right):
            pl.semaphore_signal(barrier, inc=1, device_id=(nbr,),
                                device_id_type=pl.DeviceIdType.MESH)
        pl.semaphore_wait(barrier, 2)

        for b in range(B):
            for kvl in range(2):
                g = 2 * my + kvl
                pltpu.make_async_copy(
                    k_any.at[b, :, g, :], k_heads.at[b * 2 + kvl],
                    kv_sems.at[b, kvl]).start()
                pltpu.make_async_copy(
                    v_any.at[b, :, g, :], v_heads.at[b * 2 + kvl],
                    kv_sems.at[b, 2 + kvl]).start()

        x_bf[...] = x_ref[...].astype(BF)
        wq_bf[...] = wq_ref[...].astype(BF)
        wo_bf[...] = wo_ref[...].astype(BF)

        colz = lax.broadcasted_iota(jnp.int32, (SKV, DH), 1)
        ones_col = jnp.where(colz == 0, 1.0, 0.0).astype(BF)
        v_aug[0, :, DH:] = ones_col
        v_aug[1, :, DH:] = ones_col

        def compute_chunk(b):
            rows = pl.ds(b * SQ, SQ)
            q_bf[...] = (jnp.dot(x_bf[rows, :], wq_bf[...],
                                 preferred_element_type=jnp.float32)
                         * SCALE).astype(BF)

            for kvl in range(2):
                g = 2 * my + kvl
                pltpu.make_async_copy(
                    k_any.at[b, :, g, :], k_heads.at[b * 2 + kvl],
                    kv_sems.at[b, kvl]).wait()
                pltpu.make_async_copy(
                    v_any.at[b, :, g, :], v_heads.at[b * 2 + kvl],
                    kv_sems.at[b, 2 + kvl]).wait()
                k_bf[kvl] = k_heads[b * 2 + kvl].astype(BF)
                v_aug[kvl, :, :DH] = v_heads[b * 2 + kvl].astype(BF)

            for kvl in range(2):
                cols = pl.ds(kvl * 4 * DH, 4 * DH)
                q3 = q_bf[:, cols].reshape(SQ * 4, DH)
                s3 = lax.dot_general(
                    q3, k_bf[kvl], (((1,), (1,)), ((), ())),
                    preferred_element_type=jnp.float32)
                p3 = jnp.exp(s3).astype(BF)
                o_aug = jnp.dot(p3, v_aug[kvl],
                                preferred_element_type=jnp.float32)
                o = o_aug[:, :DH] / o_aug[:, DH:DH + 1]
                attn_bf[:, cols] = o.astype(BF).reshape(SQ, 4 * DH)

            out_ref[rows, :] = jnp.dot(attn_bf[...], wo_bf[...],
                                       preferred_element_type=jnp.float32)

        compute_chunk(my)
        prev = pltpu.make_async_remote_copy(
            src_ref=out_ref.at[pl.ds(my * SQ, SQ)],
            dst_ref=rs_buf.at[0],
            send_sem=rs_send.at[0], recv_sem=rs_recv.at[0],
            device_id=(right,), device_id_type=pl.DeviceIdType.MESH,
        )
        prev.start()
        for idx in range(1, N_DEV):
            b = lax.rem(my - idx + N_DEV, N_DEV)
            compute_chunk(b)
            prev.wait()
            rows = pl.ds(b * SQ, SQ)
            out_ref[rows, :] = out_ref[rows, :] + rs_buf[idx - 1]
            if idx < N_DEV - 1:
                prev = pltpu.make_async_remote_copy(
                    src_ref=out_ref.at[rows],
                    dst_ref=rs_buf.at[idx],
                    send_sem=rs_send.at[idx], recv_sem=rs_recv.at[idx],
                    device_id=(right,), device_id_type=pl.DeviceIdType.MESH,
                )
                prev.start()

        c_own = lax.rem(my + 1, N_DEV)
        ag_bf[0] = out_ref[pl.ds(c_own * SQ, SQ), :].astype(BF)
        a_r = pltpu.make_async_remote_copy(
            src_ref=ag_bf.at[0], dst_ref=ag_bf.at[1],
            send_sem=ag_sems.at[0], recv_sem=ag_sems.at[1],
            device_id=(right,), device_id_type=pl.DeviceIdType.MESH,
        )
        a_l = pltpu.make_async_remote_copy(
            src_ref=ag_bf.at[0], dst_ref=ag_bf.at[2],
            send_sem=ag_sems.at[2], recv_sem=ag_sems.at[3],
            device_id=(left,), device_id_type=pl.DeviceIdType.MESH,
        )
        a_r.start()
        a_l.start()
        a_r.wait()
        fwd = pltpu.make_async_remote_copy(
            src_ref=ag_bf.at[1], dst_ref=ag_bf.at[3],
            send_sem=ag_sems.at[4], recv_sem=ag_sems.at[5],
            device_id=(right,), device_id_type=pl.DeviceIdType.MESH,
        )
        fwd.start()
        out_ref[pl.ds(my * SQ, SQ), :] = ag_bf[1].astype(jnp.float32)
        a_l.wait()
        c2 = lax.rem(my + 2, N_DEV)
        out_ref[pl.ds(c2 * SQ, SQ), :] = ag_bf[2].astype(jnp.float32)
        fwd.wait()
        c3 = lax.rem(my + 3, N_DEV)
        out_ref[pl.ds(c3 * SQ, SQ), :] = ag_bf[3].astype(jnp.float32)

    out = pl.pallas_call(
        body,
        out_shape=jax.ShapeDtypeStruct((B * SQ, D), jnp.float32),
        in_specs=[
            pl.BlockSpec(memory_space=pltpu.VMEM),
            pl.BlockSpec(memory_space=pltpu.VMEM),
            pl.BlockSpec(memory_space=pltpu.VMEM),
            pl.BlockSpec(memory_space=pl.ANY),
            pl.BlockSpec(memory_space=pl.ANY),
        ],
        out_specs=pl.BlockSpec(memory_space=pltpu.VMEM),
        scratch_shapes=[
            pltpu.VMEM((B * SQ, D), BF),
            pltpu.VMEM((D, D), BF),
            pltpu.VMEM((D, D), BF),
            pltpu.VMEM((B * 2, SKV, DH), jnp.float32),
            pltpu.VMEM((B * 2, SKV, DH), jnp.float32),
            pltpu.VMEM((2, SKV, DH), BF),
            pltpu.VMEM((2, SKV, 2 * DH), BF),
            pltpu.VMEM((SQ, D), BF),
            pltpu.VMEM((SQ, D), BF),
            pltpu.VMEM((N_DEV - 1, SQ, D), jnp.float32),
            pltpu.VMEM((N_DEV, SQ, D), BF),
            pltpu.SemaphoreType.DMA((B, 4)),
            pltpu.SemaphoreType.DMA((N_DEV - 1,)),
            pltpu.SemaphoreType.DMA((N_DEV - 1,)),
            pltpu.SemaphoreType.DMA((6,)),
        ],
        compiler_params=pltpu.CompilerParams(
            collective_id=0, vmem_limit_bytes=100 * 1024 * 1024),
    )(x2, Wq, Wo, K_ext, V_ext)
    return out.reshape(B, SQ, D)


# device time: 60855 ns/iter; 3.4303x vs baseline; 1.2753x over previous
import jax
import jax.numpy as jnp
from jax import lax
from jax.experimental import pallas as pl
from jax.experimental.pallas import tpu as pltpu

N_DEV = 4
B, SQ, D = 4, 256, 1024
HQ_LOCAL = 8
DH = 128
SKV = 1024
SCALE = 0.08838834764831843
BF = jnp.bfloat16


def kernel(x, Wq, Wo, K_ext, V_ext):
    x2 = x.reshape(B * SQ, D)

    def body(x_ref, wq_ref, wo_ref, k_any, v_any, out_ref,
             x_bf, wq_bf, wo_bf, k_heads, v_heads, k_bf, v_aug,
             q_bf, attn_bf, rs_out, rs_buf, ag_bf,
             kv_sems, rs_send, rs_recv, ag_sems):
        my = lax.axis_index("i")
        left = lax.rem(my + N_DEV - 1, N_DEV)
        right = lax.rem(my + 1, N_DEV)

        barrier = pltpu.get_barrier_semaphore()
        for nbr in (left, right):
            pl.semaphore_signal(barrier, inc=1, device_id=(nbr,),
                                device_id_type=pl.DeviceIdType.MESH)
        pl.semaphore_wait(barrier, 2)

        for b in range(B):
            for kvl in range(2):
                g = 2 * my + kvl
                pltpu.make_async_copy(
                    k_any.at[b, :, g, :], k_heads.at[b * 2 + kvl],
                    kv_sems.at[b, kvl]).start()
                pltpu.make_async_copy(
                    v_any.at[b, :, g, :], v_heads.at[b * 2 + kvl],
                    kv_sems.at[b, 2 + kvl]).start()

        x_bf[...] = x_ref[...].astype(BF)
        wq_bf[...] = wq_ref[...].astype(BF)
        wo_bf[...] = wo_ref[...].astype(BF)

        colz = lax.broadcasted_iota(jnp.int32, (SKV, DH), 1)
        ones_col = jnp.where(colz == 0, 1.0, 0.0).astype(BF)
        v_aug[0, :, DH:] = ones_col
        v_aug[1, :, DH:] = ones_col

        def compute_chunk(b):
            rows = pl.ds(b * SQ, SQ)
            q_bf[...] = (jnp.dot(x_bf[rows, :], wq_bf[...],
                                 preferred_element_type=jnp.float32)
                         * SCALE).astype(BF)

            for kvl in range(2):
                g = 2 * my + kvl
                pltpu.make_async_copy(
                    k_any.at[b, :, g, :], k_heads.at[b * 2 + kvl],
                    kv_sems.at[b, kvl]).wait()
                pltpu.make_async_copy(
                    v_any.at[b, :, g, :], v_heads.at[b * 2 + kvl],
                    kv_sems.at[b, 2 + kvl]).wait()
                k_bf[kvl] = k_heads[b * 2 + kvl].astype(BF)
                v_aug[kvl, :, :DH] = v_heads[b * 2 + kvl].astype(BF)

            for kvl in range(2):
                cols = pl.ds(kvl * 4 * DH, 4 * DH)
                q3 = q_bf[:, cols].reshape(SQ * 4, DH)
                s3 = lax.dot_general(
                    q3, k_bf[kvl], (((1,), (1,)), ((), ())),
                    preferred_element_type=jnp.float32)
                p3 = jnp.exp(s3).astype(BF)
                o_aug = jnp.dot(p3, v_aug[kvl],
                                preferred_element_type=jnp.float32)
                o = o_aug[:, :DH] / o_aug[:, DH:DH + 1]
                attn_bf[:, cols] = o.astype(BF).reshape(SQ, 4 * DH)

            out_ref[rows, :] = jnp.dot(attn_bf[...], wo_bf[...],
                                       preferred_element_type=jnp.float32)

        compute_chunk(my)
        rs_out[0] = out_ref[pl.ds(my * SQ, SQ), :].astype(BF)
        prev = pltpu.make_async_remote_copy(
            src_ref=rs_out.at[0],
            dst_ref=rs_buf.at[0],
            send_sem=rs_send.at[0], recv_sem=rs_recv.at[0],
            device_id=(right,), device_id_type=pl.DeviceIdType.MESH,
        )
        prev.start()
        for idx in range(1, N_DEV):
            b = lax.rem(my - idx + N_DEV, N_DEV)
            compute_chunk(b)
            prev.wait()
            rows = pl.ds(b * SQ, SQ)
            acc = out_ref[rows, :] + rs_buf[idx - 1].astype(jnp.float32)
            if idx < N_DEV - 1:
                rs_out[idx] = acc.astype(BF)
                prev = pltpu.make_async_remote_copy(
                    src_ref=rs_out.at[idx],
                    dst_ref=rs_buf.at[idx],
                    send_sem=rs_send.at[idx], recv_sem=rs_recv.at[idx],
                    device_id=(right,), device_id_type=pl.DeviceIdType.MESH,
                )
                prev.start()
            else:
                out_ref[rows, :] = acc

        c_own = lax.rem(my + 1, N_DEV)
        ag_bf[0] = out_ref[pl.ds(c_own * SQ, SQ), :].astype(BF)
        a_r = pltpu.make_async_remote_copy(
            src_ref=ag_bf.at[0], dst_ref=ag_bf.at[1],
            send_sem=ag_sems.at[0], recv_sem=ag_sems.at[1],
            device_id=(right,), device_id_type=pl.DeviceIdType.MESH,
        )
        a_l = pltpu.make_async_remote_copy(
            src_ref=ag_bf.at[0], dst_ref=ag_bf.at[2],
            send_sem=ag_sems.at[2], recv_sem=ag_sems.at[3],
            device_id=(left,), device_id_type=pl.DeviceIdType.MESH,
        )
        a_r.start()
        a_l.start()
        a_r.wait()
        fwd = pltpu.make_async_remote_copy(
            src_ref=ag_bf.at[1], dst_ref=ag_bf.at[3],
            send_sem=ag_sems.at[4], recv_sem=ag_sems.at[5],
            device_id=(right,), device_id_type=pl.DeviceIdType.MESH,
        )
        fwd.start()
        out_ref[pl.ds(my * SQ, SQ), :] = ag_bf[1].astype(jnp.float32)
        a_l.wait()
        c2 = lax.rem(my + 2, N_DEV)
        out_ref[pl.ds(c2 * SQ, SQ), :] = ag_bf[2].astype(jnp.float32)
        fwd.wait()
        c3 = lax.rem(my + 3, N_DEV)
        out_ref[pl.ds(c3 * SQ, SQ), :] = ag_bf[3].astype(jnp.float32)

    out = pl.pallas_call(
        body,
        out_shape=jax.ShapeDtypeStruct((B * SQ, D), jnp.float32),
        in_specs=[
            pl.BlockSpec(memory_space=pltpu.VMEM),
            pl.BlockSpec(memory_space=pltpu.VMEM),
            pl.BlockSpec(memory_space=pltpu.VMEM),
            pl.BlockSpec(memory_space=pl.ANY),
            pl.BlockSpec(memory_space=pl.ANY),
        ],
        out_specs=pl.BlockSpec(memory_space=pltpu.VMEM),
        scratch_shapes=[
            pltpu.VMEM((B * SQ, D), BF),
            pltpu.VMEM((D, D), BF),
            pltpu.VMEM((D, D), BF),
            pltpu.VMEM((B * 2, SKV, DH), jnp.float32),
            pltpu.VMEM((B * 2, SKV, DH), jnp.float32),
            pltpu.VMEM((2, SKV, DH), BF),
            pltpu.VMEM((2, SKV, 2 * DH), BF),
            pltpu.VMEM((SQ, D), BF),
            pltpu.VMEM((SQ, D), BF),
            pltpu.VMEM((N_DEV - 1, SQ, D), BF),
            pltpu.VMEM((N_DEV - 1, SQ, D), BF),
            pltpu.VMEM((N_DEV, SQ, D), BF),
            pltpu.SemaphoreType.DMA((B, 4)),
            pltpu.SemaphoreType.DMA((N_DEV - 1,)),
            pltpu.SemaphoreType.DMA((N_DEV - 1,)),
            pltpu.SemaphoreType.DMA((6,)),
        ],
        compiler_params=pltpu.CompilerParams(
            collective_id=0, vmem_limit_bytes=100 * 1024 * 1024),
    )(x2, Wq, Wo, K_ext, V_ext)
    return out.reshape(B, SQ, D)
